# baseline (speedup 1.0000x reference)
import functools

import jax
import jax.numpy as jnp
from jax import lax
from jax.experimental import pallas as pl
from jax.experimental.pallas import tpu as pltpu
from jax.experimental.pallas import tpu_sc as plsc

N = 10000
E = 320000
D = 128
NPAD = 10240
NC = 2
NS = 16
NW = NC * NS
EPW = E // NW
K = 128
NFULL = EPW // K
KT = EPW - NFULL * K
CHUNKS = E // K
BCH = CHUNKS // NW
XCH = CHUNKS - BCH * NW
RPT = NPAD // NS
BLK = 1024
NBLK = NPAD // BLK

_MESH = plsc.VectorSubcoreMesh(core_axis_name="c", subcore_axis_name="s")
_F32 = jnp.float32


def _worker_id():
    return lax.axis_index("s") * NC + lax.axis_index("c")


def _reg_fill(dst_ref, src_ref, src_off, n):
    for t in range(n // 16):
        dst_ref[pl.ds(t * 16, 16)] = src_ref[pl.ds(src_off + t * 16, 16)]


@functools.partial(
    pl.kernel,
    out_type=jax.ShapeDtypeStruct((2 * NPAD, 16), _F32),
    mesh=_MESH,
    scratch_types=[
        pltpu.VMEM((K, 16), _F32),
        pltpu.VMEM((64, 16), _F32),
        pltpu.VMEM((K,), jnp.int32),
        pltpu.VMEM((K,), jnp.int32),
        pltpu.VMEM((KT,), jnp.int32),
        pltpu.VMEM_SHARED((NPAD, 16), _F32),
        pltpu.SemaphoreType.DMA,
        pltpu.SemaphoreType.DMA,
    ],
)
def _deg_kernel(dst_hbm, out_hbm, ones_v, zb_v, didx_a, didx_b, didx_t,
                deg_sh, sem_a, sem_b):
    cid = lax.axis_index("c")
    sid = lax.axis_index("s")
    wid = _worker_id()

    def fill(i, _):
        ones_v[i, :] = jnp.ones((16,), _F32)
        return 0

    lax.fori_loop(0, K, fill, 0)

    def zfill(i, _):
        zb_v[i, :] = jnp.zeros((16,), _F32)
        return 0

    lax.fori_loop(0, 64, zfill, 0)

    def zcp(i, _):
        pltpu.sync_copy(zb_v, deg_sh.at[pl.ds(sid * RPT + i * 64, 64)])
        return 0

    lax.fori_loop(0, RPT // 64, zcp, 0)
    plsc.subcore_barrier()

    base = wid * EPW

    def d_start(j, buf, sem):
        pltpu.async_copy(dst_hbm.at[pl.ds(base + j * K, K)], buf, sem)

    def d_wait(j, buf, sem):
        pltpu.make_async_copy(dst_hbm.at[pl.ds(base + j * K, K)],
                              buf, sem).wait()

    d_start(0, didx_a, sem_a)

    def pair(p, _):
        j = 2 * p
        d_start(j + 1, didx_b, sem_b)
        d_wait(j, didx_a, sem_a)
        pltpu.sync_copy(ones_v, deg_sh.at[didx_a], add=True)

        @pl.when(j + 2 < NFULL)
        def _():
            d_start(j + 2, didx_a, sem_a)

        d_wait(j + 1, didx_b, sem_b)
        pltpu.sync_copy(ones_v, deg_sh.at[didx_b], add=True)
        return 0

    lax.fori_loop(0, NFULL // 2, pair, 0)
    pltpu.sync_copy(dst_hbm.at[pl.ds(base + NFULL * K, KT)], didx_t)
    pltpu.sync_copy(ones_v.at[pl.ds(0, KT)], deg_sh.at[didx_t], add=True)

    plsc.subcore_barrier()
    pltpu.sync_copy(
        deg_sh.at[pl.ds(sid * RPT, RPT)],
        out_hbm.at[pl.ds(cid * NPAD + sid * RPT, RPT)],
    )


KS = 100
SCH = E // KS
SPW = SCH // NW
DH = D // 2


@functools.partial(
    pl.kernel,
    out_type=jax.ShapeDtypeStruct((2 * NPAD, D), _F32),
    mesh=_MESH,
    scratch_types=[
        pltpu.VMEM((KS,), jnp.int32),
        pltpu.VMEM((KS,), jnp.int32),
        pltpu.VMEM((KS,), jnp.int32),
        pltpu.VMEM((KS,), jnp.int32),
        pltpu.VMEM((KS,), jnp.int32),
        pltpu.VMEM((KS,), jnp.int32),
        pltpu.VMEM((KS,), jnp.int32),
        pltpu.VMEM((KS,), jnp.int32),
        pltpu.VMEM((KS, D), _F32),
        pltpu.VMEM((KS, D), _F32),
        pltpu.VMEM((KS, D), _F32),
        pltpu.VMEM((16, D), _F32),
        pltpu.VMEM_SHARED((NPAD, D), _F32),
        pltpu.SemaphoreType.DMA,
        pltpu.SemaphoreType.DMA,
        pltpu.SemaphoreType.DMA,
        pltpu.SemaphoreType.DMA,
        pltpu.SemaphoreType.DMA,
        pltpu.SemaphoreType.DMA,
        pltpu.SemaphoreType.DMA,
        pltpu.SemaphoreType.DMA,
        pltpu.SemaphoreType.DMA,
        pltpu.SemaphoreType.DMA,
    ],
)
def _scatter_kernel(g_hbm, src_hbm, dst_hbm, out_hbm,
                    si0, si1, si2, si3, di0, di1, di2, di3,
                    rb0, rb1, rb2, zb_v, acc_sh,
                    sg0, sg1, sg2, ss0, ss1, ss2, sx0, sx1, sx2, sx3):
    cid = lax.axis_index("c")
    sid = lax.axis_index("s")
    wid = _worker_id()
    ch0 = wid * SPW
    sidx = [si0, si1, si2, si3]
    didx = [di0, di1, di2, di3]
    isem = [sx0, sx1, sx2, sx3]
    rb = [rb0, rb1, rb2]
    gsem = [sg0, sg1, sg2]
    ssem = [ss0, ss1, ss2]

    def zfill(i, _):
        for j in range(D // 16):
            zb_v[i, pl.ds(j * 16, 16)] = jnp.zeros((16,), _F32)
        return 0

    lax.fori_loop(0, 16, zfill, 0)

    def zcp(i, _):
        pltpu.sync_copy(zb_v, acc_sh.at[pl.ds(sid * RPT + i * 16, 16)])
        return 0

    lax.fori_loop(0, RPT // 16, zcp, 0)
    plsc.subcore_barrier()

    def i_start(j, q4):
        pltpu.async_copy(src_hbm.at[ch0 + j, 0], sidx[q4], isem[q4])
        pltpu.async_copy(dst_hbm.at[ch0 + j, 0], didx[q4], isem[q4])

    def i_wait(j, q4):
        pltpu.make_async_copy(src_hbm.at[ch0 + j, 0], sidx[q4],
                              isem[q4]).wait()
        pltpu.make_async_copy(dst_hbm.at[ch0 + j, 0], didx[q4],
                              isem[q4]).wait()

    def g_start(q4, q3):
        pltpu.async_copy(g_hbm.at[sidx[q4]], rb[q3], gsem[q3])

    def g_wait(q4, q3):
        pltpu.make_async_copy(g_hbm.at[sidx[q4]], rb[q3], gsem[q3]).wait()

    def s_add(q4, q3):
        pltpu.sync_copy(rb[q3], acc_sh.at[didx[q4]], add=True)

    i_start(0, 0)
    i_wait(0, 0)
    g_start(0, 0)

    def body(j, q4, q3, has_next):
        if has_next:
            i_start(j + 1, (q4 + 1) % 4)
        g_wait(q4, q3)
        if has_next:
            i_wait(j + 1, (q4 + 1) % 4)
            g_start((q4 + 1) % 4, (q3 + 1) % 3)
        s_add(q4, q3)

    body(0, 0, 0, True)

    def twelve(p, _):
        jb = 1 + 12 * p
        for r in range(12):
            body(jb + r, (1 + r) % 4, (1 + r) % 3, True)
        return 0

    lax.fori_loop(0, (SPW - 4) // 12, twelve, 0)
    body(SPW - 3, (SPW - 3) % 4, (SPW - 3) % 3, True)
    body(SPW - 2, (SPW - 2) % 4, (SPW - 2) % 3, True)
    body(SPW - 1, (SPW - 1) % 4, (SPW - 1) % 3, False)

    plsc.subcore_barrier()
    pltpu.sync_copy(
        acc_sh.at[pl.ds(sid * RPT, RPT)],
        out_hbm.at[pl.ds(cid * NPAD + sid * RPT, RPT)],
    )


def _dinv_of(d0, d1):
    deg = d0[:, :1] + d1[:, :1] + 1.0
    return lax.rsqrt(deg)


def _p_body(x_ref, w_ref, d0_ref, d1_ref, g_ref):
    dinv = _dinv_of(d0_ref[...], d1_ref[...])
    g_ref[...] = dinv * jnp.dot(x_ref[...], w_ref[...],
                                preferred_element_type=_F32)


def _c_body(s0_ref, s1_ref, g_ref, d0_ref, d1_ref, b_ref, w_ref, out_ref):
    dinv = _dinv_of(d0_ref[...], d1_ref[...])
    t = dinv * (s0_ref[...] + s1_ref[...] + g_ref[...]) + b_ref[...]
    h = jnp.where(t >= 0.0, t, 0.01 * t)
    out_ref[...] = dinv * jnp.dot(h, w_ref[...], preferred_element_type=_F32)


def _c3_body(s0_ref, s1_ref, g_ref, d0_ref, d1_ref, b_ref, out_ref):
    dinv = _dinv_of(d0_ref[...], d1_ref[...])
    t = dinv * (s0_ref[...] + s1_ref[...] + g_ref[...]) + b_ref[...]
    out_ref[...] = jnp.where(t >= 0.0, t, 0.01 * t)


def _row_spec(width):
    return pl.BlockSpec((BLK, width), lambda i: (i, 0))


def _row_spec_hi(width):
    return pl.BlockSpec((BLK, width), lambda i: (i + NBLK, 0))


_FULL_W = pl.BlockSpec((D, D), lambda i: (0, 0))
_FULL_B = pl.BlockSpec((1, D), lambda i: (0, 0))

_p_call = pl.pallas_call(
    _p_body,
    grid=(NBLK,),
    in_specs=[_row_spec(D), _FULL_W, _row_spec(16), _row_spec_hi(16)],
    out_specs=_row_spec(D),
    out_shape=jax.ShapeDtypeStruct((NPAD, D), _F32),
)

_c_call = pl.pallas_call(
    _c_body,
    grid=(NBLK,),
    in_specs=[_row_spec(D), _row_spec_hi(D), _row_spec(D),
              _row_spec(16), _row_spec_hi(16), _FULL_B, _FULL_W],
    out_specs=_row_spec(D),
    out_shape=jax.ShapeDtypeStruct((NPAD, D), _F32),
)

_c3_call = pl.pallas_call(
    _c3_body,
    grid=(NBLK,),
    in_specs=[_row_spec(D), _row_spec_hi(D), _row_spec(D),
              _row_spec(16), _row_spec_hi(16), _FULL_B],
    out_specs=_row_spec(D),
    out_shape=jax.ShapeDtypeStruct((NPAD, D), _F32),
)


def kernel(x, edge_index, W1, b1, W2, b2, W3, b3):
    src = edge_index[0]
    dst = edge_index[1]
    src3 = src.reshape(SCH, 1, KS)
    dst3 = dst.reshape(SCH, 1, KS)
    xp = jnp.zeros((NPAD, D), _F32).at[:N].set(x)
    b1r = b1.reshape(1, D)
    b2r = b2.reshape(1, D)
    b3r = b3.reshape(1, D)

    dpart = _deg_kernel(dst)
    g1 = _p_call(xp, W1, dpart, dpart)
    s1 = _scatter_kernel(g1, src3, dst3)
    g2 = _c_call(s1, s1, g1, dpart, dpart, b1r, W2)
    s2 = _scatter_kernel(g2, src3, dst3)
    g3 = _c_call(s2, s2, g2, dpart, dpart, b2r, W3)
    s3 = _scatter_kernel(g3, src3, dst3)
    out = _c3_call(s3, s3, g3, dpart, dpart, b3r)
    return out[:N]


# speedup vs baseline: 1.0879x; 1.0879x over previous
import functools

import jax
import jax.numpy as jnp
from jax import lax
from jax.experimental import pallas as pl
from jax.experimental.pallas import tpu as pltpu
from jax.experimental.pallas import tpu_sc as plsc

N = 10000
E = 320000
D = 128
NPAD = 10240
NC = 2
NS = 16
NW = NC * NS
EPW = E // NW
K = 128
NFULL = EPW // K
KT = EPW - NFULL * K
CHUNKS = E // K
BCH = CHUNKS // NW
XCH = CHUNKS - BCH * NW
RPT = NPAD // NS
BLK = 1024
NBLK = NPAD // BLK

_MESH = plsc.VectorSubcoreMesh(core_axis_name="c", subcore_axis_name="s")
_F32 = jnp.float32


def _worker_id():
    return lax.axis_index("s") * NC + lax.axis_index("c")


def _reg_fill(dst_ref, src_ref, src_off, n):
    for t in range(n // 16):
        dst_ref[pl.ds(t * 16, 16)] = src_ref[pl.ds(src_off + t * 16, 16)]


@functools.partial(
    pl.kernel,
    out_type=jax.ShapeDtypeStruct((2 * NPAD, 16), _F32),
    mesh=_MESH,
    scratch_types=[
        pltpu.VMEM((K, 16), _F32),
        pltpu.VMEM((64, 16), _F32),
        pltpu.VMEM((K,), jnp.int32),
        pltpu.VMEM((K,), jnp.int32),
        pltpu.VMEM((KT,), jnp.int32),
        pltpu.VMEM_SHARED((NPAD, 16), _F32),
        pltpu.SemaphoreType.DMA,
        pltpu.SemaphoreType.DMA,
    ],
)
def _deg_kernel(dst_hbm, out_hbm, ones_v, zb_v, didx_a, didx_b, didx_t,
                deg_sh, sem_a, sem_b):
    cid = lax.axis_index("c")
    sid = lax.axis_index("s")
    wid = _worker_id()

    def fill(i, _):
        ones_v[i, :] = jnp.ones((16,), _F32)
        return 0

    lax.fori_loop(0, K, fill, 0)

    def zfill(i, _):
        zb_v[i, :] = jnp.zeros((16,), _F32)
        return 0

    lax.fori_loop(0, 64, zfill, 0)

    def zcp(i, _):
        pltpu.sync_copy(zb_v, deg_sh.at[pl.ds(sid * RPT + i * 64, 64)])
        return 0

    lax.fori_loop(0, RPT // 64, zcp, 0)
    plsc.subcore_barrier()

    base = wid * EPW

    def d_start(j, buf, sem):
        pltpu.async_copy(dst_hbm.at[pl.ds(base + j * K, K)], buf, sem)

    def d_wait(j, buf, sem):
        pltpu.make_async_copy(dst_hbm.at[pl.ds(base + j * K, K)],
                              buf, sem).wait()

    d_start(0, didx_a, sem_a)

    def pair(p, _):
        j = 2 * p
        d_start(j + 1, didx_b, sem_b)
        d_wait(j, didx_a, sem_a)
        pltpu.sync_copy(ones_v, deg_sh.at[didx_a], add=True)

        @pl.when(j + 2 < NFULL)
        def _():
            d_start(j + 2, didx_a, sem_a)

        d_wait(j + 1, didx_b, sem_b)
        pltpu.sync_copy(ones_v, deg_sh.at[didx_b], add=True)
        return 0

    lax.fori_loop(0, NFULL // 2, pair, 0)
    pltpu.sync_copy(dst_hbm.at[pl.ds(base + NFULL * K, KT)], didx_t)
    pltpu.sync_copy(ones_v.at[pl.ds(0, KT)], deg_sh.at[didx_t], add=True)

    plsc.subcore_barrier()
    pltpu.sync_copy(
        deg_sh.at[pl.ds(sid * RPT, RPT)],
        out_hbm.at[pl.ds(cid * NPAD + sid * RPT, RPT)],
    )


KS = 100
SCH = E // KS
SPW = SCH // NW
DH = D // 2


@functools.partial(
    pl.kernel,
    out_type=jax.ShapeDtypeStruct((2 * NPAD, D), _F32),
    mesh=_MESH,
    scratch_types=[
        pltpu.VMEM((KS,), jnp.int32),
        pltpu.VMEM((KS,), jnp.int32),
        pltpu.VMEM((KS,), jnp.int32),
        pltpu.VMEM((KS,), jnp.int32),
        pltpu.VMEM((KS,), jnp.int32),
        pltpu.VMEM((KS,), jnp.int32),
        pltpu.VMEM((KS,), jnp.int32),
        pltpu.VMEM((KS,), jnp.int32),
        pltpu.VMEM((KS, D), _F32),
        pltpu.VMEM((KS, D), _F32),
        pltpu.VMEM((KS, D), _F32),
        pltpu.VMEM((16, D), _F32),
        pltpu.VMEM_SHARED((NPAD, D), _F32),
        pltpu.SemaphoreType.DMA,
        pltpu.SemaphoreType.DMA,
        pltpu.SemaphoreType.DMA,
        pltpu.SemaphoreType.DMA,
        pltpu.SemaphoreType.DMA,
        pltpu.SemaphoreType.DMA,
        pltpu.SemaphoreType.DMA,
        pltpu.SemaphoreType.DMA,
        pltpu.SemaphoreType.DMA,
        pltpu.SemaphoreType.DMA,
    ],
)
def _scatter_kernel(g_hbm, src_hbm, dst_hbm, out_hbm,
                    si0, si1, si2, si3, di0, di1, di2, di3,
                    rb0, rb1, rb2, zb_v, acc_sh,
                    sg0, sg1, sg2, ss0, ss1, ss2, sx0, sx1, sx2, sx3):
    cid = lax.axis_index("c")
    sid = lax.axis_index("s")
    wid = _worker_id()
    ch0 = wid * SPW
    sidx = [si0, si1, si2, si3]
    didx = [di0, di1, di2, di3]
    isem = [sx0, sx1, sx2, sx3]
    rb = [rb0, rb1, rb2]
    gsem = [sg0, sg1, sg2]
    ssem = [ss0, ss1, ss2]

    def zfill(i, _):
        for j in range(D // 16):
            zb_v[i, pl.ds(j * 16, 16)] = jnp.zeros((16,), _F32)
        return 0

    lax.fori_loop(0, 16, zfill, 0)

    def zcp(i, _):
        pltpu.sync_copy(zb_v, acc_sh.at[pl.ds(sid * RPT + i * 16, 16)])
        return 0

    lax.fori_loop(0, RPT // 16, zcp, 0)
    plsc.subcore_barrier()

    def i_start(j, q4):
        pltpu.async_copy(src_hbm.at[ch0 + j, 0], sidx[q4], isem[q4])
        pltpu.async_copy(dst_hbm.at[ch0 + j, 0], didx[q4], isem[q4])

    def i_wait(j, q4):
        pltpu.make_async_copy(src_hbm.at[ch0 + j, 0], sidx[q4],
                              isem[q4]).wait()
        pltpu.make_async_copy(dst_hbm.at[ch0 + j, 0], didx[q4],
                              isem[q4]).wait()

    def g_start(q4, q3):
        pltpu.async_copy(g_hbm.at[sidx[q4]], rb[q3], gsem[q3])

    def g_wait(q4, q3):
        pltpu.make_async_copy(g_hbm.at[sidx[q4]], rb[q3], gsem[q3]).wait()

    def s_add(q4, q3):
        pltpu.sync_copy(rb[q3], acc_sh.at[didx[q4]], add=True)

    i_start(0, 0)
    i_start(1, 1)
    i_wait(0, 0)
    g_start(0, 0)
    i_wait(1, 1)
    g_start(1, 1)

    def body(j, q4, q3, has_next):
        if has_next:
            i_start(j + 2, (q4 + 2) % 4)
        g_wait(q4, q3)
        if has_next:
            i_wait(j + 2, (q4 + 2) % 4)
            g_start((q4 + 2) % 4, (q3 + 2) % 3)
        s_add(q4, q3)

    body(0, 0, 0, True)
    body(1, 1, 1, True)

    def twelve(p, _):
        jb = 2 + 12 * p
        for r in range(12):
            body(jb + r, (2 + r) % 4, (2 + r) % 3, True)
        return 0

    lax.fori_loop(0, (SPW - 4) // 12, twelve, 0)
    body(SPW - 2, (SPW - 2) % 4, (SPW - 2) % 3, False)
    body(SPW - 1, (SPW - 1) % 4, (SPW - 1) % 3, False)

    plsc.subcore_barrier()
    pltpu.sync_copy(
        acc_sh.at[pl.ds(sid * RPT, RPT)],
        out_hbm.at[pl.ds(cid * NPAD + sid * RPT, RPT)],
    )


def _dinv_of(d0, d1):
    deg = d0[:, :1] + d1[:, :1] + 1.0
    return lax.rsqrt(deg)


def _p_body(x_ref, w_ref, d0_ref, d1_ref, g_ref):
    dinv = _dinv_of(d0_ref[...], d1_ref[...])
    g_ref[...] = dinv * jnp.dot(x_ref[...], w_ref[...],
                                preferred_element_type=_F32)


def _c_body(s0_ref, s1_ref, g_ref, d0_ref, d1_ref, b_ref, w_ref, out_ref):
    dinv = _dinv_of(d0_ref[...], d1_ref[...])
    t = dinv * (s0_ref[...] + s1_ref[...] + g_ref[...]) + b_ref[...]
    h = jnp.where(t >= 0.0, t, 0.01 * t)
    out_ref[...] = dinv * jnp.dot(h, w_ref[...], preferred_element_type=_F32)


def _c3_body(s0_ref, s1_ref, g_ref, d0_ref, d1_ref, b_ref, out_ref):
    dinv = _dinv_of(d0_ref[...], d1_ref[...])
    t = dinv * (s0_ref[...] + s1_ref[...] + g_ref[...]) + b_ref[...]
    out_ref[...] = jnp.where(t >= 0.0, t, 0.01 * t)


def _row_spec(width):
    return pl.BlockSpec((BLK, width), lambda i: (i, 0))


def _row_spec_hi(width):
    return pl.BlockSpec((BLK, width), lambda i: (i + NBLK, 0))


_FULL_W = pl.BlockSpec((D, D), lambda i: (0, 0))
_FULL_B = pl.BlockSpec((1, D), lambda i: (0, 0))

_p_call = pl.pallas_call(
    _p_body,
    grid=(NBLK,),
    in_specs=[_row_spec(D), _FULL_W, _row_spec(16), _row_spec_hi(16)],
    out_specs=_row_spec(D),
    out_shape=jax.ShapeDtypeStruct((NPAD, D), _F32),
)

_c_call = pl.pallas_call(
    _c_body,
    grid=(NBLK,),
    in_specs=[_row_spec(D), _row_spec_hi(D), _row_spec(D),
              _row_spec(16), _row_spec_hi(16), _FULL_B, _FULL_W],
    out_specs=_row_spec(D),
    out_shape=jax.ShapeDtypeStruct((NPAD, D), _F32),
)

_c3_call = pl.pallas_call(
    _c3_body,
    grid=(NBLK,),
    in_specs=[_row_spec(D), _row_spec_hi(D), _row_spec(D),
              _row_spec(16), _row_spec_hi(16), _FULL_B],
    out_specs=_row_spec(D),
    out_shape=jax.ShapeDtypeStruct((NPAD, D), _F32),
)


def kernel(x, edge_index, W1, b1, W2, b2, W3, b3):
    src = edge_index[0]
    dst = edge_index[1]
    src3 = src.reshape(SCH, 1, KS)
    dst3 = dst.reshape(SCH, 1, KS)
    xp = jnp.zeros((NPAD, D), _F32).at[:N].set(x)
    b1r = b1.reshape(1, D)
    b2r = b2.reshape(1, D)
    b3r = b3.reshape(1, D)

    dpart = _deg_kernel(dst)
    g1 = _p_call(xp, W1, dpart, dpart)
    s1 = _scatter_kernel(g1, src3, dst3)
    g2 = _c_call(s1, s1, g1, dpart, dpart, b1r, W2)
    s2 = _scatter_kernel(g2, src3, dst3)
    g3 = _c_call(s2, s2, g2, dpart, dpart, b2r, W3)
    s3 = _scatter_kernel(g3, src3, dst3)
    out = _c3_call(s3, s3, g3, dpart, dpart, b3r)
    return out[:N]


# speedup vs baseline: 1.1018x; 1.0128x over previous
import functools

import jax
import jax.numpy as jnp
from jax import lax
from jax.experimental import pallas as pl
from jax.experimental.pallas import tpu as pltpu
from jax.experimental.pallas import tpu_sc as plsc

N = 10000
E = 320000
D = 128
NPAD = 10240
NC = 2
NS = 16
NW = NC * NS
EPW = E // NW
K = 128
NFULL = EPW // K
KT = EPW - NFULL * K
CHUNKS = E // K
BCH = CHUNKS // NW
XCH = CHUNKS - BCH * NW
RPT = NPAD // NS
BLK = 1024
NBLK = NPAD // BLK

_MESH = plsc.VectorSubcoreMesh(core_axis_name="c", subcore_axis_name="s")
_F32 = jnp.float32


def _worker_id():
    return lax.axis_index("s") * NC + lax.axis_index("c")


def _reg_fill(dst_ref, src_ref, src_off, n):
    for t in range(n // 16):
        dst_ref[pl.ds(t * 16, 16)] = src_ref[pl.ds(src_off + t * 16, 16)]


@functools.partial(
    pl.kernel,
    out_type=jax.ShapeDtypeStruct((2 * NPAD, 16), _F32),
    mesh=_MESH,
    scratch_types=[
        pltpu.VMEM((K, 16), _F32),
        pltpu.VMEM((64, 16), _F32),
        pltpu.VMEM((K,), jnp.int32),
        pltpu.VMEM((K,), jnp.int32),
        pltpu.VMEM((KT,), jnp.int32),
        pltpu.VMEM_SHARED((NPAD, 16), _F32),
        pltpu.SemaphoreType.DMA,
        pltpu.SemaphoreType.DMA,
    ],
)
def _deg_kernel(dst_hbm, out_hbm, ones_v, zb_v, didx_a, didx_b, didx_t,
                deg_sh, sem_a, sem_b):
    cid = lax.axis_index("c")
    sid = lax.axis_index("s")
    wid = _worker_id()

    def fill(i, _):
        ones_v[i, :] = jnp.ones((16,), _F32)
        return 0

    lax.fori_loop(0, K, fill, 0)

    def zfill(i, _):
        zb_v[i, :] = jnp.zeros((16,), _F32)
        return 0

    lax.fori_loop(0, 64, zfill, 0)

    def zcp(i, _):
        pltpu.sync_copy(zb_v, deg_sh.at[pl.ds(sid * RPT + i * 64, 64)])
        return 0

    lax.fori_loop(0, RPT // 64, zcp, 0)
    plsc.subcore_barrier()

    base = wid * EPW

    def d_start(j, buf, sem):
        pltpu.async_copy(dst_hbm.at[pl.ds(base + j * K, K)], buf, sem)

    def d_wait(j, buf, sem):
        pltpu.make_async_copy(dst_hbm.at[pl.ds(base + j * K, K)],
                              buf, sem).wait()

    d_start(0, didx_a, sem_a)

    def pair(p, _):
        j = 2 * p
        d_start(j + 1, didx_b, sem_b)
        d_wait(j, didx_a, sem_a)
        pltpu.sync_copy(ones_v, deg_sh.at[didx_a], add=True)

        @pl.when(j + 2 < NFULL)
        def _():
            d_start(j + 2, didx_a, sem_a)

        d_wait(j + 1, didx_b, sem_b)
        pltpu.sync_copy(ones_v, deg_sh.at[didx_b], add=True)
        return 0

    lax.fori_loop(0, NFULL // 2, pair, 0)
    pltpu.sync_copy(dst_hbm.at[pl.ds(base + NFULL * K, KT)], didx_t)
    pltpu.sync_copy(ones_v.at[pl.ds(0, KT)], deg_sh.at[didx_t], add=True)

    plsc.subcore_barrier()
    pltpu.sync_copy(
        deg_sh.at[pl.ds(sid * RPT, RPT)],
        out_hbm.at[pl.ds(cid * NPAD + sid * RPT, RPT)],
    )


KS = 100
SCH = E // KS
SPW = SCH // NW


@functools.partial(
    pl.kernel,
    out_type=jax.ShapeDtypeStruct((2 * NPAD, D), _F32),
    mesh=_MESH,
    scratch_types=[
        pltpu.VMEM((KS,), jnp.int32),
        pltpu.VMEM((KS,), jnp.int32),
        pltpu.VMEM((KS,), jnp.int32),
        pltpu.VMEM((KS,), jnp.int32),
        pltpu.VMEM((KS,), jnp.int32),
        pltpu.VMEM((KS,), jnp.int32),
        pltpu.VMEM((KS,), jnp.int32),
        pltpu.VMEM((KS,), jnp.int32),
        pltpu.VMEM((KS, D), _F32),
        pltpu.VMEM((KS, D), _F32),
        pltpu.VMEM((16, D), _F32),
        pltpu.VMEM_SHARED((NPAD, D), _F32),
        pltpu.SemaphoreType.DMA,
        pltpu.SemaphoreType.DMA,
        pltpu.SemaphoreType.DMA,
        pltpu.SemaphoreType.DMA,
        pltpu.SemaphoreType.DMA,
        pltpu.SemaphoreType.DMA,
    ],
)
def _scatter_kernel(g_hbm, src_hbm, dst_hbm, out_hbm,
                    si0, si1, si2, si3, di0, di1, di2, di3,
                    rb0, rb1, zb_v, acc_sh,
                    sg0, sg1, sx0, sx1, sx2, sx3):
    cid = lax.axis_index("c")
    sid = lax.axis_index("s")
    wid = _worker_id()
    ch0 = wid * SPW
    sidx = [si0, si1, si2, si3]
    didx = [di0, di1, di2, di3]
    isem = [sx0, sx1, sx2, sx3]
    rb = [rb0, rb1]
    gsem = [sg0, sg1]

    def zfill(i, _):
        for j in range(D // 16):
            zb_v[i, pl.ds(j * 16, 16)] = jnp.zeros((16,), _F32)
        return 0

    lax.fori_loop(0, 16, zfill, 0)

    def zcp(i, _):
        pltpu.sync_copy(zb_v, acc_sh.at[pl.ds(sid * RPT + i * 16, 16)])
        return 0

    lax.fori_loop(0, RPT // 16, zcp, 0)
    plsc.subcore_barrier()

    def i_start(j, q4):
        pltpu.async_copy(src_hbm.at[ch0 + j, 0], sidx[q4], isem[q4])
        pltpu.async_copy(dst_hbm.at[ch0 + j, 0], didx[q4], isem[q4])

    def i_wait(j, q4):
        pltpu.make_async_copy(src_hbm.at[ch0 + j, 0], sidx[q4],
                              isem[q4]).wait()
        pltpu.make_async_copy(dst_hbm.at[ch0 + j, 0], didx[q4],
                              isem[q4]).wait()

    def g_start(q4, q2):
        pltpu.async_copy(g_hbm.at[sidx[q4]], rb[q2], gsem[q2])

    def g_wait(q4, q2):
        pltpu.make_async_copy(g_hbm.at[sidx[q4]], rb[q2], gsem[q2]).wait()

    def s_add(q4, q2):
        pltpu.sync_copy(rb[q2], acc_sh.at[didx[q4]], add=True)

    i_start(0, 0)
    i_start(1, 1)
    i_wait(0, 0)
    g_start(0, 0)
    i_wait(1, 1)
    g_start(1, 1)

    def body(j, q4, q2, has_next):
        if has_next:
            i_start(j + 2, (q4 + 2) % 4)
        g_wait(q4, q2)
        if has_next:
            i_wait(j + 2, (q4 + 2) % 4)
        s_add(q4, q2)
        if has_next:
            g_start((q4 + 2) % 4, q2)

    def quad(p, _):
        jb = 4 * p
        for r in range(4):
            body(jb + r, r, r % 2, True)
        return 0

    lax.fori_loop(0, (SPW - 4) // 4, quad, 0)
    for j in range(SPW - 4, SPW):
        body(j, j % 4, j % 2, j + 2 < SPW)

    plsc.subcore_barrier()
    pltpu.sync_copy(
        acc_sh.at[pl.ds(sid * RPT, RPT)],
        out_hbm.at[pl.ds(cid * NPAD + sid * RPT, RPT)],
    )


def _dinv_of(d0, d1):
    deg = d0[:, :1] + d1[:, :1] + 1.0
    return lax.rsqrt(deg)


def _p_body(x_ref, w_ref, d0_ref, d1_ref, g_ref):
    dinv = _dinv_of(d0_ref[...], d1_ref[...])
    g_ref[...] = dinv * jnp.dot(x_ref[...], w_ref[...],
                                preferred_element_type=_F32)


def _c_body(s0_ref, s1_ref, g_ref, d0_ref, d1_ref, b_ref, w_ref, out_ref):
    dinv = _dinv_of(d0_ref[...], d1_ref[...])
    t = dinv * (s0_ref[...] + s1_ref[...] + g_ref[...]) + b_ref[...]
    h = jnp.where(t >= 0.0, t, 0.01 * t)
    out_ref[...] = dinv * jnp.dot(h, w_ref[...], preferred_element_type=_F32)


def _c3_body(s0_ref, s1_ref, g_ref, d0_ref, d1_ref, b_ref, out_ref):
    dinv = _dinv_of(d0_ref[...], d1_ref[...])
    t = dinv * (s0_ref[...] + s1_ref[...] + g_ref[...]) + b_ref[...]
    out_ref[...] = jnp.where(t >= 0.0, t, 0.01 * t)


def _row_spec(width):
    return pl.BlockSpec((BLK, width), lambda i: (i, 0))


def _row_spec_hi(width):
    return pl.BlockSpec((BLK, width), lambda i: (i + NBLK, 0))


_FULL_W = pl.BlockSpec((D, D), lambda i: (0, 0))
_FULL_B = pl.BlockSpec((1, D), lambda i: (0, 0))

_p_call = pl.pallas_call(
    _p_body,
    grid=(NBLK,),
    in_specs=[_row_spec(D), _FULL_W, _row_spec(16), _row_spec_hi(16)],
    out_specs=_row_spec(D),
    out_shape=jax.ShapeDtypeStruct((NPAD, D), _F32),
)

_c_call = pl.pallas_call(
    _c_body,
    grid=(NBLK,),
    in_specs=[_row_spec(D), _row_spec_hi(D), _row_spec(D),
              _row_spec(16), _row_spec_hi(16), _FULL_B, _FULL_W],
    out_specs=_row_spec(D),
    out_shape=jax.ShapeDtypeStruct((NPAD, D), _F32),
)

_c3_call = pl.pallas_call(
    _c3_body,
    grid=(NBLK,),
    in_specs=[_row_spec(D), _row_spec_hi(D), _row_spec(D),
              _row_spec(16), _row_spec_hi(16), _FULL_B],
    out_specs=_row_spec(D),
    out_shape=jax.ShapeDtypeStruct((NPAD, D), _F32),
)


def kernel(x, edge_index, W1, b1, W2, b2, W3, b3):
    src = edge_index[0]
    dst = edge_index[1]
    src3 = src.reshape(SCH, 1, KS)
    dst3 = dst.reshape(SCH, 1, KS)
    xp = jnp.zeros((NPAD, D), _F32).at[:N].set(x)
    b1r = b1.reshape(1, D)
    b2r = b2.reshape(1, D)
    b3r = b3.reshape(1, D)

    dpart = _deg_kernel(dst)
    g1 = _p_call(xp, W1, dpart, dpart)
    s1 = _scatter_kernel(g1, src3, dst3)
    g2 = _c_call(s1, s1, g1, dpart, dpart, b1r, W2)
    s2 = _scatter_kernel(g2, src3, dst3)
    g3 = _c_call(s2, s2, g2, dpart, dpart, b2r, W3)
    s3 = _scatter_kernel(g3, src3, dst3)
    out = _c3_call(s3, s3, g3, dpart, dpart, b3r)
    return out[:N]
